# baseline (speedup 1.0000x reference)
import functools

import jax
import jax.numpy as jnp
from jax import lax
from jax.experimental import pallas as pl
from jax.experimental.pallas import tpu as pltpu
from jax.experimental.pallas import tpu_sc as plsc

N = 2_000_000
C = 10
NB = 15
K = 65536
SCALE = 4096.0

NC = 2
NS = 16
NT = NC * NS
ROWS_T = N // NT
CR = 2500
CW = CR * C
NCH = ROWS_T // CR
FULL_G = CR // 16
TAIL = CR - FULL_G * 16


def _sc_hist_body(fx_hbm, y_hbm, out_hbm, fx_v, y_v, hist):
    c = lax.axis_index("c")
    s = lax.axis_index("s")
    tile = c * NS + s
    zero16 = jnp.zeros((16,), jnp.float32)

    def zbody(i, carry):
        for u in range(8):
            hist[pl.ds((i * 8 + u) * 16, 16)] = zero16
        return carry

    lax.fori_loop(0, K // 128, zbody, 0)

    lane = lax.broadcasted_iota(jnp.int32, (16,), 0)
    lane10 = lane * C
    base = tile * (ROWS_T * C)

    def topk_group(idx0):
        m = plsc.load_gather(fx_v, [idx0])
        a = jnp.zeros((16,), jnp.int32)
        for j in range(1, C):
            xj = plsc.load_gather(fx_v, [idx0 + j])
            cnd = xj > m
            m = jnp.where(cnd, xj, m)
            a = jnp.where(cnd, j, a)
        my = plsc.load_gather(y_v, [idx0])
        ay = jnp.zeros((16,), jnp.int32)
        for j in range(1, C):
            yj = plsc.load_gather(y_v, [idx0 + j])
            cnd = yj > my
            my = jnp.where(cnd, yj, my)
            ay = jnp.where(cnd, j, ay)
        hit = (a == ay).astype(jnp.float32)
        bucket = jnp.minimum(K - 1, (m * K).astype(jnp.int32))
        return bucket, SCALE + hit

    def chunk_body(ci, carry):
        off = base + ci * CW
        pltpu.sync_copy(fx_hbm.at[pl.ds(off, CW)], fx_v)
        pltpu.sync_copy(y_hbm.at[pl.ds(off, CW)], y_v)

        def grp(g, carry2):
            bucket, upd = topk_group(g * (16 * C) + lane10)
            plsc.addupdate_scatter(hist, [bucket], upd)
            return carry2

        lax.fori_loop(0, FULL_G, grp, 0)
        msk = lane < TAIL
        idx0 = jnp.where(msk, FULL_G * (16 * C) + lane10, 0)
        bucket, upd = topk_group(idx0)
        plsc.addupdate_scatter(hist, [bucket], upd, mask=msk)
        return carry

    lax.fori_loop(0, NCH, chunk_body, 0)
    pltpu.sync_copy(hist, out_hbm.at[tile])


def _make_sc_hist():
    return pl.kernel(
        _sc_hist_body,
        mesh=plsc.VectorSubcoreMesh(core_axis_name="c", subcore_axis_name="s"),
        out_type=jax.ShapeDtypeStruct((NT, K), jnp.float32),
        scratch_types=[
            pltpu.VMEM((CW,), jnp.float32),
            pltpu.VMEM((CW,), jnp.float32),
            pltpu.VMEM((K,), jnp.float32),
        ],
        compiler_params=pltpu.CompilerParams(needs_layout_passes=False),
    )


def _tc_fin_body(h_ref, o_ref):
    h = jnp.sum(h_ref[...], axis=0)
    cnt = jnp.floor(h * (1.0 / SCALE))
    sumh = h - SCALE * cnt
    ri = lax.broadcasted_iota(jnp.int32, (512, 128), 0).astype(jnp.float32)
    ci = lax.broadcasted_iota(jnp.int32, (512, 128), 1).astype(jnp.float32)
    midv = (ri * 128.0 + ci + 0.5) * (1.0 / K)
    sumv = cnt * midv
    iu = lax.broadcasted_iota(jnp.int32, (128, 128), 0)
    ju = lax.broadcasted_iota(jnp.int32, (128, 128), 1)
    upper = (iu <= ju).astype(jnp.float32)
    rowpre = jnp.dot(cnt, upper, preferred_element_type=jnp.float32)
    rowtot = rowpre[:, 127:128]
    il = lax.broadcasted_iota(jnp.int32, (512, 512), 0)
    jl = lax.broadcasted_iota(jnp.int32, (512, 512), 1)
    lower = (jl < il).astype(jnp.float32)
    excl = jnp.dot(lower, rowtot, preferred_element_type=jnp.float32)
    c_incl = rowpre + excl
    mid_rank = c_incl - 0.5 * cnt
    bink = jnp.minimum(14.0, jnp.floor(mid_rank * (NB / N)))
    ce_sum = jnp.float32(0.0)
    for b in range(NB):
        mask = bink == b
        sn = jnp.sum(jnp.where(mask, cnt, 0.0))
        sfx = jnp.sum(jnp.where(mask, sumv, 0.0))
        sh = jnp.sum(jnp.where(mask, sumh, 0.0))
        safe = jnp.maximum(sn, 1.0)
        diff = (sfx - sh) / safe
        ce_sum = ce_sum + jnp.where(sn > 0, diff * diff * sn, 0.0)
    o_ref[0, 0] = jnp.sqrt(ce_sum / N)


def kernel(fx, y):
    hists = _make_sc_hist()(fx.reshape(-1), y.reshape(-1))
    out = pl.pallas_call(
        _tc_fin_body,
        out_shape=jax.ShapeDtypeStruct((1, 1), jnp.float32),
        in_specs=[pl.BlockSpec((NT, 512, 128), lambda: (0, 0, 0))],
        out_specs=pl.BlockSpec(memory_space=pltpu.SMEM),
    )(hists.reshape(NT, 512, 128))
    return out[0, 0]


# speedup vs baseline: 6.2885x; 6.2885x over previous
import functools

import jax
import jax.numpy as jnp
from jax import lax
from jax.experimental import pallas as pl
from jax.experimental.pallas import tpu as pltpu
from jax.experimental.pallas import tpu_sc as plsc

N = 2_000_000
NB = 15
K = 65536
SCALE = 4096.0
NPAD = 2_097_152
LB = 16384
GRID1 = NPAD // LB
LAST_IN_BLOCK = (N + LB - 1) // LB - 1

NC = 2
NS = 16
NT = NC * NS
PER_TILE = NPAD // NT
CH = 8192
HT = K + 16


def _tc_top_body(fx_ref, y_ref, o_ref):
    i = pl.program_id(0)
    x = fx_ref[...]
    yv = y_ref[...]
    m = x[0]
    a = jnp.zeros((LB,), jnp.int32)
    for j in range(1, 10):
        xj = x[j]
        c = xj > m
        m = jnp.where(c, xj, m)
        a = jnp.where(c, j, a)
    my = yv[0]
    ay = jnp.zeros((LB,), jnp.int32)
    for j in range(1, 10):
        yj = yv[j]
        c = yj > my
        my = jnp.where(c, yj, my)
        ay = jnp.where(c, j, ay)
    hit = (a == ay).astype(jnp.int32)
    bucket = jnp.minimum(K - 1, (m * K).astype(jnp.int32))
    row = i * LB + lax.broadcasted_iota(jnp.int32, (LB,), 0)
    o_ref[...] = jnp.where(row < N, bucket + hit * 131072, K)


def _sc_hist_body(words_hbm, out_hbm, in_v0, in_v1, hist, sem0, sem1):
    c = lax.axis_index("c")
    s = lax.axis_index("s")
    tile = c * NS + s
    base = tile * PER_TILE
    zero16 = jnp.zeros((16,), jnp.float32)
    bufs = (in_v0, in_v1)
    sems = (sem0, sem1)
    nchunk = PER_TILE // CH
    handles = [None, None]
    handles[0] = pltpu.async_copy(words_hbm.at[pl.ds(base, CH)], in_v0, sem0)

    def zbody(i, carry):
        for u in range(8):
            hist[pl.ds((i * 8 + u) * 16, 16)] = zero16
        return carry

    lax.fori_loop(0, HT // 128, zbody, 0)
    for t in range(nchunk):
        if t + 1 < nchunk:
            handles[(t + 1) % 2] = pltpu.async_copy(
                words_hbm.at[pl.ds(base + (t + 1) * CH, CH)],
                bufs[(t + 1) % 2], sems[(t + 1) % 2])
        handles[t % 2].wait()
        in_v = bufs[t % 2]

        def body(i, carry):
            for u in range(4):
                w = in_v[pl.ds((i * 4 + u) * 16, 16)]
                b16 = lax.bitwise_and(w, 0x1FFFF)
                h16 = lax.shift_right_logical(w, 17)
                plsc.addupdate_scatter(
                    hist, [b16], SCALE + h16.astype(jnp.float32))
            return carry

        lax.fori_loop(0, CH // 64, body, 0)
    pltpu.sync_copy(hist.at[pl.ds(0, K)], out_hbm.at[tile])


def _make_sc_hist():
    return pl.kernel(
        _sc_hist_body,
        mesh=plsc.VectorSubcoreMesh(core_axis_name="c", subcore_axis_name="s"),
        out_type=jax.ShapeDtypeStruct((NT, K), jnp.float32),
        scratch_types=[
            pltpu.VMEM((CH,), jnp.int32),
            pltpu.VMEM((CH,), jnp.int32),
            pltpu.VMEM((HT,), jnp.float32),
            pltpu.SemaphoreType.DMA,
            pltpu.SemaphoreType.DMA,
        ],
        compiler_params=pltpu.CompilerParams(needs_layout_passes=False),
    )


def _tc_fin_body(h_ref, o_ref):
    h = jnp.sum(h_ref[...], axis=0)
    cnt = jnp.floor(h * (1.0 / SCALE))
    sumh = h - SCALE * cnt
    ri = lax.broadcasted_iota(jnp.int32, (512, 128), 0).astype(jnp.float32)
    ci = lax.broadcasted_iota(jnp.int32, (512, 128), 1).astype(jnp.float32)
    midv = (ri * 128.0 + ci + 0.5) * (1.0 / K)
    sumv = cnt * midv
    iu = lax.broadcasted_iota(jnp.int32, (128, 128), 0)
    ju = lax.broadcasted_iota(jnp.int32, (128, 128), 1)
    upper = (iu <= ju).astype(jnp.float32)
    rowpre = jnp.dot(cnt, upper, preferred_element_type=jnp.float32)
    rowtot = rowpre[:, 127:128]
    il = lax.broadcasted_iota(jnp.int32, (512, 512), 0)
    jl = lax.broadcasted_iota(jnp.int32, (512, 512), 1)
    lower = (jl < il).astype(jnp.float32)
    excl = jnp.dot(lower, rowtot, preferred_element_type=jnp.float32)
    c_incl = rowpre + excl
    mid_rank = c_incl - 0.5 * cnt
    bink = jnp.minimum(14.0, jnp.floor(mid_rank * (NB / N)))
    ce_sum = jnp.float32(0.0)
    for b in range(NB):
        mask = bink == b
        sn = jnp.sum(jnp.where(mask, cnt, 0.0))
        sfx = jnp.sum(jnp.where(mask, sumv, 0.0))
        sh = jnp.sum(jnp.where(mask, sumh, 0.0))
        safe = jnp.maximum(sn, 1.0)
        diff = (sfx - sh) / safe
        ce_sum = ce_sum + jnp.where(sn > 0, diff * diff * sn, 0.0)
    o_ref[0, 0] = jnp.sqrt(ce_sum / N)


def kernel(fx, y):
    fxt = fx.T
    yt = y.T
    words = pl.pallas_call(
        _tc_top_body,
        grid=(GRID1,),
        out_shape=jax.ShapeDtypeStruct((NPAD,), jnp.int32),
        in_specs=[
            pl.BlockSpec((10, LB), lambda i: (0, jnp.minimum(i, LAST_IN_BLOCK))),
            pl.BlockSpec((10, LB), lambda i: (0, jnp.minimum(i, LAST_IN_BLOCK))),
        ],
        out_specs=pl.BlockSpec((LB,), lambda i: (i,)),
    )(fxt, yt)
    hists = _make_sc_hist()(words)
    out = pl.pallas_call(
        _tc_fin_body,
        out_shape=jax.ShapeDtypeStruct((1, 1), jnp.float32),
        in_specs=[pl.BlockSpec((NT, 512, 128), lambda: (0, 0, 0))],
        out_specs=pl.BlockSpec(memory_space=pltpu.SMEM),
    )(hists.reshape(NT, 512, 128))
    return out[0, 0]


# speedup vs baseline: 7.4799x; 1.1894x over previous
import functools

import jax
import jax.numpy as jnp
from jax import lax
from jax.experimental import pallas as pl
from jax.experimental.pallas import tpu as pltpu
from jax.experimental.pallas import tpu_sc as plsc

N = 2_000_000
NB = 15
K = 65536
SCALE = 4096.0
NPAD = 2_097_152
LB = 16384
GRID1 = NPAD // LB
LAST_IN_BLOCK = (N + LB - 1) // LB - 1

NC = 2
NS = 16
NT = NC * NS
PER_TILE = NPAD // NT
CH = 8192
HT = K + 16


def _tc_top_body(fx_ref, y_ref, o_ref):
    i = pl.program_id(0)
    x = fx_ref[...]
    yv = y_ref[...]
    m = x[0]
    a = jnp.zeros((LB,), jnp.int32)
    for j in range(1, 10):
        xj = x[j]
        c = xj > m
        m = jnp.where(c, xj, m)
        a = jnp.where(c, j, a)
    my = yv[0]
    ay = jnp.zeros((LB,), jnp.int32)
    for j in range(1, 10):
        yj = yv[j]
        c = yj > my
        my = jnp.where(c, yj, my)
        ay = jnp.where(c, j, ay)
    hit = (a == ay).astype(jnp.int32)
    bucket = jnp.minimum(K - 1, (m * K).astype(jnp.int32))
    row = i * LB + lax.broadcasted_iota(jnp.int32, (LB,), 0)
    o_ref[...] = jnp.where(row < N, bucket + hit * 131072,
                           K + lax.bitwise_and(row, 15))


def _sc_hist_body(words_hbm, out_hbm, in_v0, in_v1, hist, sem0, sem1):
    c = lax.axis_index("c")
    s = lax.axis_index("s")
    tile = c * NS + s
    base = tile * PER_TILE
    zero16 = jnp.zeros((16,), jnp.float32)
    bufs = (in_v0, in_v1)
    sems = (sem0, sem1)
    nchunk = PER_TILE // CH
    handles = [None, None]
    handles[0] = pltpu.async_copy(words_hbm.at[pl.ds(base, CH)], in_v0, sem0)

    def zbody(i, carry):
        for u in range(8):
            hist[pl.ds((i * 8 + u) * 16, 16)] = zero16
        return carry

    lax.fori_loop(0, HT // 128, zbody, 0)
    for t in range(nchunk):
        if t + 1 < nchunk:
            handles[(t + 1) % 2] = pltpu.async_copy(
                words_hbm.at[pl.ds(base + (t + 1) * CH, CH)],
                bufs[(t + 1) % 2], sems[(t + 1) % 2])
        handles[t % 2].wait()
        in_v = bufs[t % 2]

        def body(i, carry):
            for u in range(4):
                w = in_v[pl.ds((i * 4 + u) * 16, 16)]
                b16 = lax.bitwise_and(w, 0x1FFFF)
                h16 = lax.shift_right_logical(w, 17)
                plsc.addupdate_scatter(
                    hist, [b16], SCALE + h16.astype(jnp.float32))
            return carry

        lax.fori_loop(0, CH // 64, body, 0)
    pltpu.sync_copy(hist.at[pl.ds(0, K)], out_hbm.at[tile])


def _make_sc_hist():
    return pl.kernel(
        _sc_hist_body,
        mesh=plsc.VectorSubcoreMesh(core_axis_name="c", subcore_axis_name="s"),
        out_type=jax.ShapeDtypeStruct((NT, K), jnp.float32),
        scratch_types=[
            pltpu.VMEM((CH,), jnp.int32),
            pltpu.VMEM((CH,), jnp.int32),
            pltpu.VMEM((HT,), jnp.float32),
            pltpu.SemaphoreType.DMA,
            pltpu.SemaphoreType.DMA,
        ],
        compiler_params=pltpu.CompilerParams(needs_layout_passes=False),
    )


def _tc_fin_body(h_ref, o_ref):
    h = jnp.sum(h_ref[...], axis=0)
    cnt = jnp.floor(h * (1.0 / SCALE))
    sumh = h - SCALE * cnt
    ri = lax.broadcasted_iota(jnp.int32, (512, 128), 0).astype(jnp.float32)
    ci = lax.broadcasted_iota(jnp.int32, (512, 128), 1).astype(jnp.float32)
    midv = (ri * 128.0 + ci + 0.5) * (1.0 / K)
    sumv = cnt * midv
    iu = lax.broadcasted_iota(jnp.int32, (128, 128), 0)
    ju = lax.broadcasted_iota(jnp.int32, (128, 128), 1)
    upper = (iu <= ju).astype(jnp.float32)
    rowpre = jnp.dot(cnt, upper, preferred_element_type=jnp.float32)
    rowtot = rowpre[:, 127:128]
    il = lax.broadcasted_iota(jnp.int32, (512, 512), 0)
    jl = lax.broadcasted_iota(jnp.int32, (512, 512), 1)
    lower = (jl < il).astype(jnp.float32)
    excl = jnp.dot(lower, rowtot, preferred_element_type=jnp.float32)
    c_incl = rowpre + excl
    mid_rank = c_incl - 0.5 * cnt
    bink = jnp.minimum(14.0, jnp.floor(mid_rank * (NB / N)))
    ce_sum = jnp.float32(0.0)
    for b in range(NB):
        mask = bink == b
        sn = jnp.sum(jnp.where(mask, cnt, 0.0))
        sfx = jnp.sum(jnp.where(mask, sumv, 0.0))
        sh = jnp.sum(jnp.where(mask, sumh, 0.0))
        safe = jnp.maximum(sn, 1.0)
        diff = (sfx - sh) / safe
        ce_sum = ce_sum + jnp.where(sn > 0, diff * diff * sn, 0.0)
    o_ref[0, 0] = jnp.sqrt(ce_sum / N)


def kernel(fx, y):
    fxt = fx.T
    yt = y.T
    words = pl.pallas_call(
        _tc_top_body,
        grid=(GRID1,),
        out_shape=jax.ShapeDtypeStruct((NPAD,), jnp.int32),
        in_specs=[
            pl.BlockSpec((10, LB), lambda i: (0, jnp.minimum(i, LAST_IN_BLOCK))),
            pl.BlockSpec((10, LB), lambda i: (0, jnp.minimum(i, LAST_IN_BLOCK))),
        ],
        out_specs=pl.BlockSpec((LB,), lambda i: (i,)),
    )(fxt, yt)
    hists = _make_sc_hist()(words)
    out = pl.pallas_call(
        _tc_fin_body,
        out_shape=jax.ShapeDtypeStruct((1, 1), jnp.float32),
        in_specs=[pl.BlockSpec((NT, 512, 128), lambda: (0, 0, 0))],
        out_specs=pl.BlockSpec(memory_space=pltpu.SMEM),
    )(hists.reshape(NT, 512, 128))
    return out[0, 0]
